# baseline (speedup 1.0000x reference)
import functools

import jax
import jax.numpy as jnp
from jax import lax
from jax.experimental import pallas as pl
from jax.experimental.pallas import tpu as pltpu
from jax.experimental.pallas import tpu_sc as plsc

_INFO = plsc.get_sparse_core_info()
_NC = _INFO.num_cores
_NS = _INFO.num_subcores
_NW = _NC * _NS

_SKP = 80


def _embed(x, table):
    R, C = x.shape
    V, D = table.shape
    TR = D // 8
    TC = R // 128
    tc_per_w = TC // _NW
    CH = 256
    n_units = C * 2
    mesh = plsc.VectorSubcoreMesh(core_axis_name="c", subcore_axis_name="s")

    xin4 = x.reshape(TC, 128, C // 8, 8).transpose(2, 0, 3, 1)

    @functools.partial(
        pl.kernel,
        mesh=mesh,
        out_type=jax.ShapeDtypeStruct((C, TR, TC, 8, 128), jnp.float32),
        scratch_types=[
            pltpu.VMEM((2, 2, 128), jnp.int32),
            pltpu.VMEM((2, CH, D), jnp.float32),
            pltpu.VMEM((CH * _SKP,), jnp.float32),
            pltpu.VMEM((2, 2 * TR, 8, 128), jnp.float32),
            pltpu.SemaphoreType.DMA,
            pltpu.SemaphoreType.DMA,
            pltpu.SemaphoreType.DMA,
            pltpu.SemaphoreType.DMA,
        ],
        compiler_params=pltpu.CompilerParams(
            use_tc_tiling_on_sc=False, needs_layout_passes=False),
    )
    def k(table_hbm, x4_hbm, out_hbm, idx_v, rows_v, sk, t5, g0, g1, w0, w1):
        wid = lax.axis_index("s") * _NC + lax.axis_index("c")
        tc0 = wid * tc_per_w
        gsem = (g0, g1)
        wsem = (w0, w1)

        c81 = lax.iota(jnp.int32, 16) * (_SKP + 1)

        def jparts(u):
            j = u >> 1
            return j >> 3, j & 7, u & 1

        def idx_src(u):
            jt, js, h = jparts(u)
            return x4_hbm.at[jt, pl.ds(tc0 + 2 * h, 2), js]

        def gather(b):
            for itx in (0, 1):
                pltpu.async_copy(
                    table_hbm.at[idx_v.at[b, itx]],
                    rows_v.at[b, pl.ds(128 * itx, 128)], gsem[b])

        def gather_wait(b):
            for itx in (0, 1):
                pltpu.make_async_copy(
                    table_hbm.at[idx_v.at[b, itx]],
                    rows_v.at[b, pl.ds(128 * itx, 128)], gsem[b]).wait()

        def out_dst(u, tcx):
            jt, js, h = jparts(u)
            return out_hbm.at[(jt << 3) + js, :, tc0 + 2 * h + tcx]

        def t5_src(b, tcx):
            return t5.at[b, pl.ds(8 * tcx, 8)]

        def write(u, b):
            for tcx in (0, 1):
                pltpu.async_copy(t5_src(b, tcx), out_dst(u, tcx), wsem[b])

        def write_wait(u, b):
            for tcx in (0, 1):
                pltpu.make_async_copy(
                    t5_src(b, tcx), out_dst(u, tcx), wsem[b]).wait()

        def transpose(b):
            def rr_body(rr, _):
                rb = rr * 16 * _SKP
                for ri in range(16):
                    base = rb + ri * (_SKP + 1)
                    for d0 in (0, 16, 32, 48):
                        sk[pl.ds(base + d0, 16)] = (
                            rows_v[b, rr * 16 + ri, pl.ds(d0, 16)])
                return 0
            lax.fori_loop(0, CH // 16, rr_body, 0)

            def lb_body(lb, _):
                for tcx in (0, 1):
                    h = c81 + ((tcx * 128 + lb * 16) * _SKP)
                    for d in range(D):
                        vec = plsc.load_gather(sk, [h + d])
                        t5[b, (tcx << 3) + (d >> 3), d & 7,
                           pl.ds(lb * 16, 16)] = vec
                return 0
            lax.fori_loop(0, 8, lb_body, 0)

        pltpu.sync_copy(idx_src(0), idx_v.at[0])
        gather(0)

        def pair_body(p, _):
            for b in (0, 1):
                u = 2 * p + b
                nb = 1 - b
                gather_wait(b)
                if b == 0:
                    pltpu.sync_copy(idx_src(u + 1), idx_v.at[nb])
                    gather(nb)
                else:
                    @pl.when(p < C - 1)
                    def _():
                        pltpu.sync_copy(idx_src(u + 1), idx_v.at[nb])
                        gather(nb)
                @pl.when(p > 0)
                def _():
                    write_wait(u - 2, b)
                transpose(b)
                write(u, b)
            return 0

        lax.fori_loop(0, C, pair_body, 0)
        write_wait(n_units - 2, 0)
        write_wait(n_units - 1, 1)

    out5 = k(table, xin4)
    return out5.transpose(2, 4, 0, 1, 3).reshape(R, C, D)


def kernel(x, table):
    return _embed(x.astype(jnp.int32), table)


# speedup vs baseline: 1.6273x; 1.6273x over previous
import functools

import jax
import jax.numpy as jnp
from jax import lax
from jax.experimental import pallas as pl
from jax.experimental.pallas import tpu as pltpu
from jax.experimental.pallas import tpu_sc as plsc

_INFO = plsc.get_sparse_core_info()
_NC = _INFO.num_cores
_NS = _INFO.num_subcores
_NW = _NC * _NS

_LP = 133
_LS = 136


def _embed(x, table):
    R, C = x.shape
    V, D = table.shape
    TR = D // 8
    TC = R // 128
    tc_per_w = TC // _NW
    CH = 256
    n_units = C * 2
    mesh = plsc.VectorSubcoreMesh(core_axis_name="c", subcore_axis_name="s")

    xin4 = x.reshape(TC, 128, C // 8, 8).transpose(2, 0, 3, 1)

    @functools.partial(
        pl.kernel,
        mesh=mesh,
        out_type=jax.ShapeDtypeStruct((C, TR, TC, 8, 128), jnp.float32),
        scratch_types=[
            pltpu.VMEM((2, 2, 128), jnp.int32),
            pltpu.VMEM((2, CH, D), jnp.float32),
            pltpu.VMEM((2, 2 * TR, 8, _LP), jnp.float32),
            pltpu.SemaphoreType.DMA,
            pltpu.SemaphoreType.DMA,
            pltpu.SemaphoreType.DMA,
            pltpu.SemaphoreType.DMA,
        ],
        compiler_params=pltpu.CompilerParams(
            use_tc_tiling_on_sc=False, needs_layout_passes=False),
    )
    def k(table_hbm, x4_hbm, out_hbm, idx_v, rows_v, t5p, g0, g1, w0, w1):
        wid = lax.axis_index("s") * _NC + lax.axis_index("c")
        tc0 = wid * tc_per_w
        gsem = (g0, g1)
        wsem = (w0, w1)

        iota = lax.iota(jnp.int32, 16)
        zc = jnp.zeros((16,), jnp.int32)
        c_flat = (iota >> 3) * (8 * _LS) + (iota & 7) * _LS

        def jparts(u):
            j = u >> 1
            return j >> 3, j & 7, u & 1

        def idx_src(u):
            jt, js, h = jparts(u)
            return x4_hbm.at[jt, pl.ds(tc0 + 2 * h, 2), js]

        def gather(b):
            for itx in (0, 1):
                pltpu.async_copy(
                    table_hbm.at[idx_v.at[b, itx]],
                    rows_v.at[b, pl.ds(128 * itx, 128)], gsem[b])

        def gather_wait(b):
            for itx in (0, 1):
                pltpu.make_async_copy(
                    table_hbm.at[idx_v.at[b, itx]],
                    rows_v.at[b, pl.ds(128 * itx, 128)], gsem[b]).wait()

        def out_dst(u, tcx):
            jt, js, h = jparts(u)
            return out_hbm.at[(jt << 3) + js, :, tc0 + 2 * h + tcx]

        def t5_src(b, tcx):
            return t5p.at[b, pl.ds(8 * tcx, 8), :, pl.ds(0, 128)]

        def write(u, b):
            for tcx in (0, 1):
                pltpu.async_copy(t5_src(b, tcx), out_dst(u, tcx), wsem[b])

        def write_wait(u, b):
            for tcx in (0, 1):
                pltpu.make_async_copy(
                    t5_src(b, tcx), out_dst(u, tcx), wsem[b]).wait()

        def transpose(b):
            def rr_body(rr, _):
                r0 = rr * 32
                for ri in range(32):
                    r = r0 + ri
                    sbase = ((r >> 7) << 3) * (8 * _LS) + (r & 127)
                    for d0 in (0, 16, 32, 48):
                        v = rows_v[b, r, pl.ds(d0, 16)]
                        flat = c_flat + (sbase + (d0 >> 3) * (8 * _LS))
                        plsc.store_scatter(t5p.at[b], [zc, zc, flat], v)
                return 0
            lax.fori_loop(0, CH // 32, rr_body, 0)

        pltpu.sync_copy(idx_src(0), idx_v.at[0])
        gather(0)

        def pair_body(p, _):
            for b in (0, 1):
                u = 2 * p + b
                nb = 1 - b
                gather_wait(b)
                if b == 0:
                    pltpu.sync_copy(idx_src(u + 1), idx_v.at[nb])
                    gather(nb)
                else:
                    @pl.when(p < C - 1)
                    def _():
                        pltpu.sync_copy(idx_src(u + 1), idx_v.at[nb])
                        gather(nb)
                @pl.when(p > 0)
                def _():
                    write_wait(u - 2, b)
                transpose(b)
                write(u, b)
            return 0

        lax.fori_loop(0, C, pair_body, 0)
        write_wait(n_units - 2, 0)
        write_wait(n_units - 1, 1)

    out5 = k(table, xin4)
    return out5.transpose(2, 4, 0, 1, 3).reshape(R, C, D)


def kernel(x, table):
    return _embed(x.astype(jnp.int32), table)
